# baseline (speedup 1.0000x reference)
import functools

import jax
import jax.numpy as jnp
from jax import lax
from jax.experimental import pallas as pl
from jax.experimental.pallas import tpu as pltpu
from jax.experimental.pallas import tpu_sc as plsc


def _rowsum_body(x_ref, o_ref):
    o_ref[...] = jnp.sum(x_ref[...], axis=1)


def _gram_w_body(n_edges, a_ref, w1_ref, b1_ref, g_ref, be_ref,
                 w1k_ref, w2k_ref, b2_ref, o_ref, m_acc, c_acc):
    phase = pl.program_id(0)
    j = pl.program_id(1)
    a = a_ref[...]

    @pl.when(phase == 0)
    def _gram():
        @pl.when(j == 0)
        def _init():
            m_acc[...] = jnp.zeros_like(m_acc)
            c_acc[...] = jnp.zeros_like(c_acc)

        m_acc[...] += lax.dot_general(
            a, a, (((0,), (0,)), ((), ())), preferred_element_type=jnp.float32
        )
        c_acc[...] += jnp.sum(a, axis=0, keepdims=True)
        o_ref[...] = jnp.zeros_like(o_ref)

    @pl.when(phase == 1)
    def _edge_w():
        m = m_acc[...]
        c = c_acc[...]
        g16 = m[0:16, 0:16]
        cs16 = c[:, 0:16]
        for k in range(1, 8):
            g16 = g16 + m[16 * k:16 * (k + 1), 16 * k:16 * (k + 1)]
            cs16 = cs16 + c[:, 16 * k:16 * (k + 1)]
        inv_e = 1.0 / float(n_edges)
        w1 = w1_ref[...]
        t = jnp.dot(cs16 * inv_e, w1, preferred_element_type=jnp.float32)
        mu = t + b1_ref[...]
        p = jnp.dot(g16 * inv_e, w1, preferred_element_type=jnp.float32)
        q = jnp.sum(w1 * p, axis=0, keepdims=True)
        var = q - t * t
        scale = g_ref[...] * lax.rsqrt(var + 1e-5)
        b1f = (b1_ref[...] - mu) * scale + be_ref[...]
        scale128 = jnp.concatenate([scale] * 8, axis=1)
        b1f128 = jnp.concatenate([b1f] * 8, axis=1)
        hh = jnp.dot(a, w1k_ref[...] * scale128,
                     preferred_element_type=jnp.float32) + b1f128
        hh = jnp.maximum(hh, 0.0)
        o_ref[...] = jnp.dot(hh, w2k_ref[...],
                             preferred_element_type=jnp.float32) + b2_ref[0, 0]


def _fin_body(inv_d, p_ref, o_ref):
    o_ref[...] = jnp.sum(p_ref[...], axis=0) * inv_d


def kernel(x, edge_index, batch, edge_attr, W1, b1, gamma, beta, W2, b2):
    n, d = x.shape
    e, de = edge_attr.shape
    g = 64
    h = W1.shape[1]
    pack = 128 // de
    r = e // pack

    s = pl.pallas_call(
        _rowsum_body,
        grid=(1,),
        in_specs=[pl.BlockSpec((n, d), lambda i: (0, 0))],
        out_specs=pl.BlockSpec((n,), lambda i: (0,)),
        out_shape=jax.ShapeDtypeStruct((n,), jnp.float32),
    )(x)

    ar = edge_attr.reshape(r, pack * de)
    br = min(4000, r)
    eye = jnp.eye(pack, dtype=jnp.float32)
    w1k = jnp.kron(eye, W1)
    w2k = jnp.kron(eye, W2)
    small = lambda a: pl.BlockSpec(a.shape, lambda i, j: tuple(0 for _ in a.shape))
    b1r = b1.reshape(1, h)
    gr = gamma.reshape(1, h)
    ber = beta.reshape(1, h)
    b2r = b2.reshape(1, 1)
    w_packed = pl.pallas_call(
        functools.partial(_gram_w_body, e),
        grid=(2, r // br),
        in_specs=[
            pl.BlockSpec((br, pack * de), lambda i, j: (j, 0)),
            small(W1), small(b1r), small(gr), small(ber),
            small(w1k), small(w2k), small(b2r),
        ],
        out_specs=pl.BlockSpec((br, pack), lambda i, j: (j, 0)),
        out_shape=jax.ShapeDtypeStruct((r, pack), jnp.float32),
        scratch_shapes=[
            pltpu.VMEM((pack * de, pack * de), jnp.float32),
            pltpu.VMEM((1, pack * de), jnp.float32),
        ],
    )(ar, W1, b1r, gr, ber, w1k, w2k, b2r)
    w = w_packed.reshape(e)

    nc, ns, lanes = 2, 16, 16
    nw = nc * ns
    epw = e // nw
    mesh = plsc.VectorSubcoreMesh(
        core_axis_name="c", subcore_axis_name="s", num_cores=nc)

    @functools.partial(
        pl.kernel,
        out_type=jax.ShapeDtypeStruct((nw, lanes * g), jnp.float32),
        mesh=mesh,
        compiler_params=pltpu.CompilerParams(
            needs_layout_passes=False, use_tc_tiling_on_sc=False),
        scratch_types=[
            pltpu.VMEM((epw,), jnp.float32),
            pltpu.VMEM((epw,), jnp.int32),
            pltpu.VMEM((epw,), jnp.int32),
            pltpu.VMEM((n,), jnp.float32),
            pltpu.VMEM((n,), jnp.int32),
            pltpu.VMEM((lanes * g,), jnp.float32),
        ],
    )
    def _sc_seg(w_hbm, ei_hbm, s_hbm, b_hbm, out_hbm,
                w_v, src_v, dst_v, s_v, b_v, acc_v):
        wid = lax.axis_index("s") * nc + lax.axis_index("c")
        base = wid * epw
        pltpu.sync_copy(s_hbm, s_v)
        pltpu.sync_copy(b_hbm, b_v)
        pltpu.sync_copy(w_hbm.at[pl.ds(base, epw)], w_v)
        pltpu.sync_copy(ei_hbm.at[0, pl.ds(base, epw)], src_v)
        pltpu.sync_copy(ei_hbm.at[1, pl.ds(base, epw)], dst_v)
        for jj in range(g):
            acc_v[pl.ds(jj * lanes, lanes)] = jnp.zeros((lanes,), jnp.float32)
        lane = lax.iota(jnp.int32, lanes)

        def body(i, carry):
            off = i * lanes
            srcv = src_v[pl.ds(off, lanes)]
            dstv = dst_v[pl.ds(off, lanes)]
            wv = w_v[pl.ds(off, lanes)]
            sv = plsc.load_gather(s_v, [srcv])
            bg = plsc.load_gather(b_v, [dstv])
            idx = lane * g + bg
            plsc.addupdate_scatter(acc_v, [idx], wv * sv)
            return carry

        lax.fori_loop(0, epw // lanes, body, 0)
        pltpu.sync_copy(acc_v, out_hbm.at[wid])

    partials = _sc_seg(w, edge_index, s, batch)

    p2 = partials.reshape(nw * lanes, g)
    out = pl.pallas_call(
        functools.partial(_fin_body, 1.0 / float(d)),
        grid=(1,),
        in_specs=[pl.BlockSpec((nw * lanes, g), lambda i: (0, 0))],
        out_specs=pl.BlockSpec((g,), lambda i: (0,)),
        out_shape=jax.ShapeDtypeStruct((g,), jnp.float32),
    )(p2)
    return out


# speedup vs baseline: 3.1718x; 3.1718x over previous
import functools

import jax
import jax.numpy as jnp
from jax import lax
from jax.experimental import pallas as pl
from jax.experimental.pallas import tpu as pltpu
from jax.experimental.pallas import tpu_sc as plsc


def _mlp_body(n_edges, be, n_nodes, at_ref, x_ref, w1_ref, w2r_ref, g_ref,
              be_ref, b2_ref, o_ref, s_ref, sh_acc, s2_acc):
    phase = pl.program_id(0)
    j = pl.program_id(1)
    a = at_ref[...]
    ht = lax.dot_general(w1_ref[...], a, (((0,), (0,)), ((), ())),
                         preferred_element_type=jnp.float32)

    @pl.when(phase == 0)
    def _stats():
        @pl.when(j == 0)
        def _init():
            sh_acc[...] = jnp.zeros_like(sh_acc)
            s2_acc[...] = jnp.zeros_like(s2_acc)
            ones_d = jnp.ones((x_ref.shape[1], 1), jnp.float32)
            sv = lax.dot_general(x_ref[...], ones_d, (((1,), (0,)), ((), ())),
                                 preferred_element_type=jnp.float32)
            s_ref[...] = sv.reshape(n_nodes)

        sh_acc[...] += jnp.sum(ht, axis=1, keepdims=True)
        s2_acc[...] += jnp.sum(ht * ht, axis=1, keepdims=True)

    @pl.when(phase == 1)
    def _edge_w():
        inv_e = 1.0 / float(n_edges)
        mu = sh_acc[...] * inv_e
        var = s2_acc[...] * inv_e - mu * mu
        gcol = jnp.transpose(g_ref[...].reshape(1, 16))
        becol = jnp.transpose(be_ref[...].reshape(1, 16))
        scale = gcol * lax.rsqrt(var + 1e-5)
        off = becol - mu * scale
        hn = jnp.maximum(ht * scale + off, 0.0)
        wv = lax.dot_general(w2r_ref[...], hn, (((1,), (0,)), ((), ())),
                             preferred_element_type=jnp.float32)
        wv = wv + b2_ref[0]
        o_ref[pl.ds(j * be, be)] = wv.reshape(be)


def _fin_body(inv_d, p_ref, o_ref):
    o_ref[...] = jnp.sum(p_ref[...], axis=0) * inv_d


def kernel(x, edge_index, batch, edge_attr, W1, b1, gamma, beta, W2, b2):
    n, d = x.shape
    e, de = edge_attr.shape
    g = 64
    h = W1.shape[1]
    del b1

    at = edge_attr.T
    w2r = W2.reshape(1, h)
    be_blk = min(32000, e)
    nblk = e // be_blk
    small = lambda a: pl.BlockSpec(a.shape, lambda i, j: tuple(0 for _ in a.shape))
    w, s = pl.pallas_call(
        functools.partial(_mlp_body, e, be_blk, n),
        grid=(2, nblk),
        in_specs=[
            pl.BlockSpec((de, be_blk), lambda i, j: (0, j)),
            pl.BlockSpec((n, d), lambda i, j: (0, 0)),
            small(W1), small(w2r), small(gamma), small(beta), small(b2),
        ],
        out_specs=[
            pl.BlockSpec((e,), lambda i, j: (0,)),
            pl.BlockSpec((n,), lambda i, j: (0,)),
        ],
        out_shape=[
            jax.ShapeDtypeStruct((e,), jnp.float32),
            jax.ShapeDtypeStruct((n,), jnp.float32),
        ],
        scratch_shapes=[
            pltpu.VMEM((de, 1), jnp.float32),
            pltpu.VMEM((de, 1), jnp.float32),
        ],
    )(at, x, W1, w2r, gamma, beta, b2)

    nc, ns, lanes = 2, 16, 16
    nw = nc * ns
    epw = e // nw
    mesh = plsc.VectorSubcoreMesh(
        core_axis_name="c", subcore_axis_name="s", num_cores=nc)

    @functools.partial(
        pl.kernel,
        out_type=jax.ShapeDtypeStruct((nw * lanes, g), jnp.float32),
        mesh=mesh,
        compiler_params=pltpu.CompilerParams(
            needs_layout_passes=False, use_tc_tiling_on_sc=False),
        scratch_types=[
            pltpu.VMEM((epw,), jnp.float32),
            pltpu.VMEM((epw,), jnp.int32),
            pltpu.VMEM((epw,), jnp.int32),
            pltpu.VMEM((n,), jnp.float32),
            pltpu.VMEM((n,), jnp.int32),
            pltpu.VMEM((lanes, g), jnp.float32),
            pltpu.SemaphoreType.DMA,
        ],
    )
    def _sc_seg(w_hbm, ei_hbm, s_hbm, b_hbm, out_hbm,
                w_v, src_v, dst_v, s_v, b_v, acc_v, sem):
        wid = lax.axis_index("s") * nc + lax.axis_index("c")
        base = wid * epw
        c1 = pltpu.async_copy(s_hbm, s_v, sem)
        c2 = pltpu.async_copy(b_hbm, b_v, sem)
        c3 = pltpu.async_copy(w_hbm.at[pl.ds(base, epw)], w_v, sem)
        c4 = pltpu.async_copy(ei_hbm.at[0, pl.ds(base, epw)], src_v, sem)
        c5 = pltpu.async_copy(ei_hbm.at[1, pl.ds(base, epw)], dst_v, sem)
        zero = jnp.zeros((lanes,), jnp.float32)
        for row in range(lanes):
            for col in range(g // lanes):
                acc_v[row, pl.ds(col * lanes, lanes)] = zero
        c1.wait()
        c2.wait()
        c3.wait()
        c4.wait()
        c5.wait()
        lane = lax.iota(jnp.int32, lanes)

        @plsc.parallel_loop(0, epw // lanes, unroll=8)
        def _loop(i):
            off = i * lanes
            srcv = src_v[pl.ds(off, lanes)]
            dstv = dst_v[pl.ds(off, lanes)]
            wv = w_v[pl.ds(off, lanes)]
            sv = plsc.load_gather(s_v, [srcv])
            bg = plsc.load_gather(b_v, [dstv])
            plsc.addupdate_scatter(acc_v, [lane, bg], wv * sv)

        pltpu.sync_copy(acc_v, out_hbm.at[pl.ds(wid * lanes, lanes), :])

    partials = _sc_seg(w, edge_index, s, batch)

    out = pl.pallas_call(
        functools.partial(_fin_body, 1.0 / float(d)),
        grid=(1,),
        in_specs=[pl.BlockSpec((nw * lanes, g), lambda i: (0, 0))],
        out_specs=pl.BlockSpec((g,), lambda i: (0,)),
        out_shape=jax.ShapeDtypeStruct((g,), jnp.float32),
    )(partials)
    return out


# speedup vs baseline: 3.5943x; 1.1332x over previous
import functools

import jax
import jax.numpy as jnp
from jax import lax
from jax.experimental import pallas as pl
from jax.experimental.pallas import tpu as pltpu
from jax.experimental.pallas import tpu_sc as plsc


def _mlp_body(n_edges, be, n_nodes, at_ref, x_ref, w1_ref, w2r_ref, g_ref,
              be_ref, b2_ref, o_ref, s_ref, sh_acc, s2_acc, ht_s):
    phase = pl.program_id(0)
    j = pl.program_id(1)

    @pl.when(phase == 0)
    def _stats():
        @pl.when(j == 0)
        def _init():
            sh_acc[...] = jnp.zeros_like(sh_acc)
            s2_acc[...] = jnp.zeros_like(s2_acc)
            ones_d = jnp.ones((x_ref.shape[1], 1), jnp.float32)
            sv = lax.dot_general(x_ref[...], ones_d, (((1,), (0,)), ((), ())),
                                 preferred_element_type=jnp.float32)
            s_ref[...] = sv.reshape(n_nodes)

        a = at_ref[...]
        ht = lax.dot_general(w1_ref[...], a, (((0,), (0,)), ((), ())),
                             preferred_element_type=jnp.float32)
        ht_s[:, pl.ds(j * be, be)] = ht
        sh_acc[...] += jnp.sum(ht, axis=1, keepdims=True)
        s2_acc[...] += jnp.sum(ht * ht, axis=1, keepdims=True)

    @pl.when(phase == 1)
    def _edge_w():
        inv_e = 1.0 / float(n_edges)
        mu = sh_acc[...] * inv_e
        var = s2_acc[...] * inv_e - mu * mu
        gcol = jnp.transpose(g_ref[...].reshape(1, 16))
        becol = jnp.transpose(be_ref[...].reshape(1, 16))
        scale = gcol * lax.rsqrt(var + 1e-5)
        off = becol - mu * scale
        ht = ht_s[:, pl.ds(j * be, be)]
        hn = jnp.maximum(ht * scale + off, 0.0)
        wv = lax.dot_general(w2r_ref[...], hn, (((1,), (0,)), ((), ())),
                             preferred_element_type=jnp.float32)
        wv = wv + b2_ref[0]
        o_ref[pl.ds(j * be, be)] = wv.reshape(be)


def _fin_body(inv_d, p_ref, o_ref):
    o_ref[...] = jnp.sum(p_ref[...], axis=0) * inv_d


def kernel(x, edge_index, batch, edge_attr, W1, b1, gamma, beta, W2, b2):
    n, d = x.shape
    e, de = edge_attr.shape
    g = 64
    h = W1.shape[1]
    del b1

    at = edge_attr.T
    w2r = W2.reshape(1, h)
    be_blk = min(32000, e)
    nblk = e // be_blk
    small = lambda a: pl.BlockSpec(a.shape, lambda i, j: tuple(0 for _ in a.shape))
    w, s = pl.pallas_call(
        functools.partial(_mlp_body, e, be_blk, n),
        grid=(2, nblk),
        in_specs=[
            pl.BlockSpec((de, be_blk), lambda i, j: (0, j * (1 - i))),
            pl.BlockSpec((n, d), lambda i, j: (0, 0)),
            small(W1), small(w2r), small(gamma), small(beta), small(b2),
        ],
        out_specs=[
            pl.BlockSpec((e,), lambda i, j: (0,)),
            pl.BlockSpec((n,), lambda i, j: (0,)),
        ],
        out_shape=[
            jax.ShapeDtypeStruct((e,), jnp.float32),
            jax.ShapeDtypeStruct((n,), jnp.float32),
        ],
        scratch_shapes=[
            pltpu.VMEM((de, 1), jnp.float32),
            pltpu.VMEM((de, 1), jnp.float32),
            pltpu.VMEM((de, e), jnp.float32),
        ],
    )(at, x, W1, w2r, gamma, beta, b2)

    nc, ns, lanes = 2, 16, 16
    nw = nc * ns
    epw = e // nw
    mesh = plsc.VectorSubcoreMesh(
        core_axis_name="c", subcore_axis_name="s", num_cores=nc)

    @functools.partial(
        pl.kernel,
        out_type=jax.ShapeDtypeStruct((nw * lanes, g), jnp.float32),
        mesh=mesh,
        compiler_params=pltpu.CompilerParams(
            needs_layout_passes=False, use_tc_tiling_on_sc=False),
        scratch_types=[
            pltpu.VMEM((epw,), jnp.float32),
            pltpu.VMEM((epw,), jnp.int32),
            pltpu.VMEM((epw,), jnp.int32),
            pltpu.VMEM((n,), jnp.float32),
            pltpu.VMEM((n,), jnp.int32),
            pltpu.VMEM((lanes, g), jnp.float32),
            pltpu.SemaphoreType.DMA,
        ],
    )
    def _sc_seg(w_hbm, ei_hbm, s_hbm, b_hbm, out_hbm,
                w_v, src_v, dst_v, s_v, b_v, acc_v, sem):
        wid = lax.axis_index("s") * nc + lax.axis_index("c")
        base = wid * epw
        c1 = pltpu.async_copy(s_hbm, s_v, sem)
        c2 = pltpu.async_copy(b_hbm, b_v, sem)
        c3 = pltpu.async_copy(w_hbm.at[pl.ds(base, epw)], w_v, sem)
        c4 = pltpu.async_copy(ei_hbm.at[0, pl.ds(base, epw)], src_v, sem)
        c5 = pltpu.async_copy(ei_hbm.at[1, pl.ds(base, epw)], dst_v, sem)
        zero = jnp.zeros((lanes,), jnp.float32)
        for row in range(lanes):
            for col in range(g // lanes):
                acc_v[row, pl.ds(col * lanes, lanes)] = zero
        c1.wait()
        c2.wait()
        c3.wait()
        c4.wait()
        c5.wait()
        lane = lax.iota(jnp.int32, lanes)

        @plsc.parallel_loop(0, epw // lanes, unroll=8)
        def _loop(i):
            off = i * lanes
            srcv = src_v[pl.ds(off, lanes)]
            dstv = dst_v[pl.ds(off, lanes)]
            wv = w_v[pl.ds(off, lanes)]
            sv = plsc.load_gather(s_v, [srcv])
            bg = plsc.load_gather(b_v, [dstv])
            plsc.addupdate_scatter(acc_v, [lane, bg], wv * sv)

        pltpu.sync_copy(acc_v, out_hbm.at[pl.ds(wid * lanes, lanes), :])

    partials = _sc_seg(w, edge_index, s, batch)

    out = pl.pallas_call(
        functools.partial(_fin_body, 1.0 / float(d)),
        grid=(1,),
        in_specs=[pl.BlockSpec((nw * lanes, g), lambda i: (0, 0))],
        out_specs=pl.BlockSpec((g,), lambda i: (0,)),
        out_shape=jax.ShapeDtypeStruct((g,), jnp.float32),
    )(partials)
    return out


# speedup vs baseline: 3.9819x; 1.1078x over previous
import functools

import jax
import jax.numpy as jnp
from jax import lax
from jax.experimental import pallas as pl
from jax.experimental.pallas import tpu as pltpu
from jax.experimental.pallas import tpu_sc as plsc


def _mlp_body(n_edges, be, n_nodes, at_ref, x_ref, w1_ref, w2r_ref, g_ref,
              be_ref, b2_ref, o_ref, s_ref, sh_acc, s2_acc, ht_s):
    phase = pl.program_id(0)
    j = pl.program_id(1)

    @pl.when(phase == 0)
    def _stats():
        @pl.when(j == 0)
        def _init():
            sh_acc[...] = jnp.zeros_like(sh_acc)
            s2_acc[...] = jnp.zeros_like(s2_acc)
            ones_d = jnp.ones((x_ref.shape[1], 1), jnp.float32)
            sv = lax.dot_general(x_ref[...], ones_d, (((1,), (0,)), ((), ())),
                                 preferred_element_type=jnp.float32)
            s_ref[...] = sv.reshape(n_nodes)

        a = at_ref[...]
        ht = lax.dot_general(w1_ref[...], a, (((0,), (0,)), ((), ())),
                             preferred_element_type=jnp.float32)
        ht_s[:, pl.ds(j * be, be)] = ht
        sh_acc[...] += jnp.sum(ht, axis=1, keepdims=True)
        s2_acc[...] += jnp.sum(ht * ht, axis=1, keepdims=True)

    @pl.when(phase == 1)
    def _edge_w():
        inv_e = 1.0 / float(n_edges)
        mu = sh_acc[...] * inv_e
        var = s2_acc[...] * inv_e - mu * mu
        gcol = jnp.transpose(g_ref[...].reshape(1, 16))
        becol = jnp.transpose(be_ref[...].reshape(1, 16))
        scale = gcol * lax.rsqrt(var + 1e-5)
        off = becol - mu * scale
        ht = ht_s[:, pl.ds(j * be, be)]
        hn = jnp.maximum(ht * scale + off, 0.0)
        wv = lax.dot_general(w2r_ref[...], hn, (((1,), (0,)), ((), ())),
                             preferred_element_type=jnp.float32)
        wv = wv + b2_ref[0]
        o_ref[pl.ds(j * be, be)] = wv.reshape(be)


def _fin_body(inv_d, g, p_ref, o_ref):
    o_ref[...] = jnp.sum(p_ref[...], axis=0)[:g] * inv_d


def kernel(x, edge_index, batch, edge_attr, W1, b1, gamma, beta, W2, b2):
    n, d = x.shape
    e, de = edge_attr.shape
    g = 64
    h = W1.shape[1]
    del b1

    at = edge_attr.T
    w2r = W2.reshape(1, h)
    be_blk = min(64000, e)
    nblk = e // be_blk
    small = lambda a: pl.BlockSpec(a.shape, lambda i, j: tuple(0 for _ in a.shape))
    w, s = pl.pallas_call(
        functools.partial(_mlp_body, e, be_blk, n),
        grid=(2, nblk),
        in_specs=[
            pl.BlockSpec((de, be_blk), lambda i, j: (0, j * (1 - i))),
            pl.BlockSpec((n, d), lambda i, j: (0, 0)),
            small(W1), small(w2r), small(gamma), small(beta), small(b2),
        ],
        out_specs=[
            pl.BlockSpec((e,), lambda i, j: (0,)),
            pl.BlockSpec((n,), lambda i, j: (0,)),
        ],
        out_shape=[
            jax.ShapeDtypeStruct((e,), jnp.float32),
            jax.ShapeDtypeStruct((n,), jnp.float32),
        ],
        scratch_shapes=[
            pltpu.VMEM((de, 1), jnp.float32),
            pltpu.VMEM((de, 1), jnp.float32),
            pltpu.VMEM((de, e), jnp.float32),
        ],
    )(at, x, W1, w2r, gamma, beta, b2)

    nc, ns, lanes = 2, 16, 16
    nw = nc * ns
    epw = e // nw
    mesh = plsc.VectorSubcoreMesh(
        core_axis_name="c", subcore_axis_name="s", num_cores=nc)

    @functools.partial(
        pl.kernel,
        out_type=jax.ShapeDtypeStruct((nw * lanes, 2 * g), jnp.float32),
        mesh=mesh,
        compiler_params=pltpu.CompilerParams(
            needs_layout_passes=False, use_tc_tiling_on_sc=False),
        scratch_types=[
            pltpu.VMEM((epw,), jnp.float32),
            pltpu.VMEM((epw,), jnp.int32),
            pltpu.VMEM((epw,), jnp.int32),
            pltpu.VMEM((n,), jnp.float32),
            pltpu.VMEM((n,), jnp.int32),
            pltpu.VMEM((lanes, 2 * g), jnp.float32),
            pltpu.SemaphoreType.DMA,
        ],
    )
    def _sc_seg(w_hbm, ei_hbm, s_hbm, b_hbm, out_hbm,
                w_v, src_v, dst_v, s_v, b_v, acc_v, sem):
        wid = lax.axis_index("s") * nc + lax.axis_index("c")
        base = wid * epw
        c1 = pltpu.async_copy(s_hbm, s_v, sem)
        c2 = pltpu.async_copy(b_hbm, b_v, sem)
        c3 = pltpu.async_copy(w_hbm.at[pl.ds(base, epw)], w_v, sem)
        c4 = pltpu.async_copy(ei_hbm.at[0, pl.ds(base, epw)], src_v, sem)
        c5 = pltpu.async_copy(ei_hbm.at[1, pl.ds(base, epw)], dst_v, sem)
        zero = jnp.zeros((lanes,), jnp.float32)
        for row in range(lanes):
            for col in range(2 * g // lanes):
                acc_v[row, pl.ds(col * lanes, lanes)] = zero
        c1.wait()
        c2.wait()
        c3.wait()
        c4.wait()
        c5.wait()
        lane = lax.iota(jnp.int32, lanes)

        @plsc.parallel_loop(0, epw // lanes, unroll=16)
        def _loop(i):
            off = i * lanes
            srcv = src_v[pl.ds(off, lanes)]
            dstv = dst_v[pl.ds(off, lanes)]
            wv = w_v[pl.ds(off, lanes)]
            sv = plsc.load_gather(s_v, [srcv])
            bg = plsc.load_gather(b_v, [dstv])
            plsc.addupdate_scatter(acc_v, [lane, bg], wv * sv)

        pltpu.sync_copy(acc_v, out_hbm.at[pl.ds(wid * lanes, lanes), :])

    partials = _sc_seg(w, edge_index, s, batch)

    out = pl.pallas_call(
        functools.partial(_fin_body, 1.0 / float(d), g),
        grid=(1,),
        in_specs=[pl.BlockSpec((nw * lanes, 2 * g), lambda i: (0, 0))],
        out_specs=pl.BlockSpec((g,), lambda i: (0,)),
        out_shape=jax.ShapeDtypeStruct((g,), jnp.float32),
    )(partials)
    return out


# speedup vs baseline: 4.0036x; 1.0054x over previous
import functools

import jax
import jax.numpy as jnp
from jax import lax
from jax.experimental import pallas as pl
from jax.experimental.pallas import tpu as pltpu
from jax.experimental.pallas import tpu_sc as plsc


def _mlp_body(n_edges, be, n_nodes, at_ref, x_ref, w1_ref, w2r_ref, g_ref,
              be_ref, b2_ref, o_ref, s_ref, sh_acc, s2_acc, ht_s):
    phase = pl.program_id(0)
    j = pl.program_id(1)

    @pl.when(phase == 0)
    def _stats():
        @pl.when(j == 0)
        def _init():
            sh_acc[...] = jnp.zeros_like(sh_acc)
            s2_acc[...] = jnp.zeros_like(s2_acc)
            ones_d = jnp.ones((x_ref.shape[1], 1), jnp.float32)
            sv = lax.dot_general(x_ref[...], ones_d, (((1,), (0,)), ((), ())),
                                 preferred_element_type=jnp.float32)
            s_ref[...] = sv.reshape(n_nodes)

        a = at_ref[...]
        ht = lax.dot_general(w1_ref[...], a, (((0,), (0,)), ((), ())),
                             preferred_element_type=jnp.float32)
        ht_s[:, pl.ds(j * be, be)] = ht
        sh_acc[...] += jnp.sum(ht, axis=1, keepdims=True)
        s2_acc[...] += jnp.sum(ht * ht, axis=1, keepdims=True)

    @pl.when(phase == 1)
    def _edge_w():
        inv_e = 1.0 / float(n_edges)
        mu = sh_acc[...] * inv_e
        var = s2_acc[...] * inv_e - mu * mu
        gcol = jnp.transpose(g_ref[...].reshape(1, 16))
        becol = jnp.transpose(be_ref[...].reshape(1, 16))
        scale = gcol * lax.rsqrt(var + 1e-5)
        off = becol - mu * scale
        ht = ht_s[:, pl.ds(j * be, be)]
        hn = jnp.maximum(ht * scale + off, 0.0)
        wv = lax.dot_general(w2r_ref[...], hn, (((1,), (0,)), ((), ())),
                             preferred_element_type=jnp.float32)
        wv = wv + b2_ref[0]
        o_ref[pl.ds(j * be, be)] = wv.reshape(be)


def _fin_body(inv_d, g, p_ref, o_ref):
    o_ref[...] = jnp.sum(p_ref[...], axis=0)[:g] * inv_d


def kernel(x, edge_index, batch, edge_attr, W1, b1, gamma, beta, W2, b2):
    n, d = x.shape
    e, de = edge_attr.shape
    g = 64
    h = W1.shape[1]
    del b1

    at = edge_attr.T
    w2r = W2.reshape(1, h)
    be_blk = min(80000, e)
    nblk = e // be_blk
    small = lambda a: pl.BlockSpec(a.shape, lambda i, j: tuple(0 for _ in a.shape))
    w, s = pl.pallas_call(
        functools.partial(_mlp_body, e, be_blk, n),
        grid=(2, nblk),
        in_specs=[
            pl.BlockSpec((de, be_blk), lambda i, j: (0, j * (1 - i))),
            pl.BlockSpec((n, d), lambda i, j: (0, 0)),
            small(W1), small(w2r), small(gamma), small(beta), small(b2),
        ],
        out_specs=[
            pl.BlockSpec((e,), lambda i, j: (0,)),
            pl.BlockSpec((n,), lambda i, j: (0,)),
        ],
        out_shape=[
            jax.ShapeDtypeStruct((e,), jnp.float32),
            jax.ShapeDtypeStruct((n,), jnp.float32),
        ],
        scratch_shapes=[
            pltpu.VMEM((de, 1), jnp.float32),
            pltpu.VMEM((de, 1), jnp.float32),
            pltpu.VMEM((de, e), jnp.float32),
        ],
    )(at, x, W1, w2r, gamma, beta, b2)

    nc, ns, lanes = 2, 16, 16
    nw = nc * ns
    epw = e // nw
    mesh = plsc.VectorSubcoreMesh(
        core_axis_name="c", subcore_axis_name="s", num_cores=nc)

    @functools.partial(
        pl.kernel,
        out_type=jax.ShapeDtypeStruct((nw * lanes, 2 * g), jnp.float32),
        mesh=mesh,
        compiler_params=pltpu.CompilerParams(
            needs_layout_passes=False, use_tc_tiling_on_sc=False),
        scratch_types=[
            pltpu.VMEM((epw,), jnp.float32),
            pltpu.VMEM((epw,), jnp.int32),
            pltpu.VMEM((epw,), jnp.int32),
            pltpu.VMEM((n,), jnp.float32),
            pltpu.VMEM((n,), jnp.int32),
            pltpu.VMEM((lanes, 2 * g), jnp.float32),
            pltpu.SemaphoreType.DMA,
        ],
    )
    def _sc_seg(w_hbm, ei_hbm, s_hbm, b_hbm, out_hbm,
                w_v, src_v, dst_v, s_v, b_v, acc_v, sem):
        wid = lax.axis_index("s") * nc + lax.axis_index("c")
        base = wid * epw
        c1 = pltpu.async_copy(s_hbm, s_v, sem)
        c2 = pltpu.async_copy(b_hbm, b_v, sem)
        c3 = pltpu.async_copy(w_hbm.at[pl.ds(base, epw)], w_v, sem)
        c4 = pltpu.async_copy(ei_hbm.at[0, pl.ds(base, epw)], src_v, sem)
        c5 = pltpu.async_copy(ei_hbm.at[1, pl.ds(base, epw)], dst_v, sem)
        zero = jnp.zeros((lanes,), jnp.float32)
        for row in range(lanes):
            for col in range(2 * g // lanes):
                acc_v[row, pl.ds(col * lanes, lanes)] = zero
        c1.wait()
        c2.wait()
        c3.wait()
        c4.wait()
        c5.wait()
        lane = lax.iota(jnp.int32, lanes)

        @plsc.parallel_loop(0, epw // lanes, unroll=16)
        def _loop(i):
            off = i * lanes
            srcv = src_v[pl.ds(off, lanes)]
            dstv = dst_v[pl.ds(off, lanes)]
            wv = w_v[pl.ds(off, lanes)]
            sv = plsc.load_gather(s_v, [srcv])
            bg = plsc.load_gather(b_v, [dstv])
            plsc.addupdate_scatter(acc_v, [lane, bg], wv * sv)

        pltpu.sync_copy(acc_v, out_hbm.at[pl.ds(wid * lanes, lanes), :])

    partials = _sc_seg(w, edge_index, s, batch)

    out = pl.pallas_call(
        functools.partial(_fin_body, 1.0 / float(d), g),
        grid=(1,),
        in_specs=[pl.BlockSpec((nw * lanes, 2 * g), lambda i: (0, 0))],
        out_specs=pl.BlockSpec((g,), lambda i: (0,)),
        out_shape=jax.ShapeDtypeStruct((g,), jnp.float32),
    )(partials)
    return out


# speedup vs baseline: 4.4255x; 1.1054x over previous
import functools

import jax
import jax.numpy as jnp
from jax import lax
from jax.experimental import pallas as pl
from jax.experimental.pallas import tpu as pltpu
from jax.experimental.pallas import tpu_sc as plsc


def _mlp_body(n_edges, be, n_nodes, at_ref, x_ref, w1_ref, w2r_ref, g_ref,
              be_ref, b2_ref, o_ref, s_ref, sh_acc, s2_acc, ht_s):
    phase = pl.program_id(0)
    j = pl.program_id(1)

    @pl.when(phase == 0)
    def _stats():
        @pl.when(j == 0)
        def _init():
            sh_acc[...] = jnp.zeros_like(sh_acc)
            s2_acc[...] = jnp.zeros_like(s2_acc)
            ones_d = jnp.ones((x_ref.shape[1], 1), jnp.float32)
            sv = lax.dot_general(x_ref[...], ones_d, (((1,), (0,)), ((), ())),
                                 preferred_element_type=jnp.float32)
            s_ref[...] = sv.reshape(n_nodes)

        a = at_ref[...]
        ht = lax.dot_general(w1_ref[...], a, (((0,), (0,)), ((), ())),
                             preferred_element_type=jnp.float32)
        ht_s[:, pl.ds(j * be, be)] = ht
        sh_acc[...] += jnp.sum(ht, axis=1, keepdims=True)
        s2_acc[...] += jnp.sum(ht * ht, axis=1, keepdims=True)

    @pl.when(phase == 1)
    def _edge_w():
        inv_e = 1.0 / float(n_edges)
        mu = sh_acc[...] * inv_e
        var = s2_acc[...] * inv_e - mu * mu
        gcol = jnp.transpose(g_ref[...].reshape(1, 16))
        becol = jnp.transpose(be_ref[...].reshape(1, 16))
        scale = gcol * lax.rsqrt(var + 1e-5)
        off = becol - mu * scale
        ht = ht_s[:, pl.ds(j * be, be)]
        hn = jnp.maximum(ht * scale + off, 0.0)
        wv = lax.dot_general(w2r_ref[...], hn, (((1,), (0,)), ((), ())),
                             preferred_element_type=jnp.float32)
        wv = wv + b2_ref[0]
        o_ref[pl.ds(j * be, be)] = wv.reshape(be)


def _fin_body(inv_d, g, p_ref, o_ref):
    o_ref[...] = jnp.sum(p_ref[...], axis=0)[:g] * inv_d


def kernel(x, edge_index, batch, edge_attr, W1, b1, gamma, beta, W2, b2):
    n, d = x.shape
    e, de = edge_attr.shape
    g = 64
    h = W1.shape[1]
    del b1

    at = edge_attr.T
    w2r = W2.reshape(1, h)
    be_blk = min(80000, e)
    nblk = e // be_blk
    small = lambda a: pl.BlockSpec(a.shape, lambda i, j: tuple(0 for _ in a.shape))
    w, s = pl.pallas_call(
        functools.partial(_mlp_body, e, be_blk, n),
        grid=(2, nblk),
        in_specs=[
            pl.BlockSpec((de, be_blk), lambda i, j: (0, j * (1 - i))),
            pl.BlockSpec((n, d), lambda i, j: (0, 0)),
            small(W1), small(w2r), small(gamma), small(beta), small(b2),
        ],
        out_specs=[
            pl.BlockSpec((e,), lambda i, j: (0,)),
            pl.BlockSpec((n,), lambda i, j: (0,)),
        ],
        out_shape=[
            jax.ShapeDtypeStruct((e,), jnp.float32),
            jax.ShapeDtypeStruct((n,), jnp.float32),
        ],
        scratch_shapes=[
            pltpu.VMEM((de, 1), jnp.float32),
            pltpu.VMEM((de, 1), jnp.float32),
            pltpu.VMEM((de, e), jnp.float32),
        ],
    )(at, x, W1, w2r, gamma, beta, b2)

    nc, ns, lanes = 2, 16, 16
    nw = nc * ns
    nt = e // 128
    base_t = nt // nw
    rem_t = nt % nw
    tmax = base_t + (1 if rem_t else 0)
    ei3 = edge_index.reshape(2, nt, 128).swapaxes(0, 1)
    mesh = plsc.VectorSubcoreMesh(
        core_axis_name="c", subcore_axis_name="s", num_cores=nc)

    @functools.partial(
        pl.kernel,
        out_type=jax.ShapeDtypeStruct((nw * lanes, 2 * g), jnp.float32),
        mesh=mesh,
        compiler_params=pltpu.CompilerParams(
            needs_layout_passes=False, use_tc_tiling_on_sc=False),
        scratch_types=[
            pltpu.VMEM((tmax * 128,), jnp.float32),
            pltpu.VMEM((tmax, 2, 128), jnp.int32),
            pltpu.VMEM((n,), jnp.float32),
            pltpu.VMEM((n,), jnp.int32),
            pltpu.VMEM((lanes, 2 * g), jnp.float32),
            pltpu.SemaphoreType.DMA,
        ],
    )
    def _sc_seg(w_hbm, ei_hbm, s_hbm, b_hbm, out_hbm,
                w_v, ei_v, s_v, b_v, acc_v, sem):
        wid = lax.axis_index("s") * nc + lax.axis_index("c")
        cnt = base_t + jnp.where(wid >= nw - rem_t, 1, 0)
        t0 = wid * base_t + jnp.maximum(wid - (nw - rem_t), 0)
        c1 = pltpu.async_copy(s_hbm, s_v, sem)
        c2 = pltpu.async_copy(b_hbm, b_v, sem)
        c3 = pltpu.async_copy(w_hbm.at[pl.ds(t0 * 128, tmax * 128)], w_v, sem)
        c4 = pltpu.async_copy(ei_hbm.at[pl.ds(t0, tmax)], ei_v, sem)
        zero = jnp.zeros((lanes,), jnp.float32)
        for row in range(lanes):
            for col in range(2 * g // lanes):
                acc_v[row, pl.ds(col * lanes, lanes)] = zero
        c1.wait()
        c2.wait()
        c3.wait()
        c4.wait()
        lane = lax.iota(jnp.int32, lanes)

        @plsc.parallel_loop(0, cnt * (128 // lanes), unroll=16)
        def _loop(i):
            t = i // (128 // lanes)
            l16 = (i % (128 // lanes)) * lanes
            srcv = ei_v[t, 0, pl.ds(l16, lanes)]
            dstv = ei_v[t, 1, pl.ds(l16, lanes)]
            wv = w_v[pl.ds(i * lanes, lanes)]
            sv = plsc.load_gather(s_v, [srcv])
            bg = plsc.load_gather(b_v, [dstv])
            plsc.addupdate_scatter(acc_v, [lane, bg], wv * sv)

        pltpu.sync_copy(acc_v, out_hbm.at[pl.ds(wid * lanes, lanes), :])

    partials = _sc_seg(w, ei3, s, batch)

    out = pl.pallas_call(
        functools.partial(_fin_body, 1.0 / float(d), g),
        grid=(1,),
        in_specs=[pl.BlockSpec((nw * lanes, 2 * g), lambda i: (0, 0))],
        out_specs=pl.BlockSpec((g,), lambda i: (0,)),
        out_shape=jax.ShapeDtypeStruct((g,), jnp.float32),
    )(partials)
    return out
